# Initial kernel scaffold; baseline (speedup 1.0000x reference)
import jax
import jax.numpy as jnp
from jax.experimental import pallas as pl


def kernel(h, x, edge_index, mask_ligand, W_e1, b_e1, W_e2, b_e2, W_inf, b_inf, W_x1, b_x1, W_x2, W_n1, b_n1, W_n2, b_n2):
    raise NotImplementedError("write your pallas kernel here")


# speedup vs baseline: 4.8681x; 4.8681x over previous
import jax
import jax.numpy as jnp
from jax import lax
from jax.experimental import pallas as pl
from jax.experimental.pallas import tpu as pltpu
from jax.experimental.pallas import tpu_sc as plsc

N = 10000
E = 320000
D = 128
NC, NS, L = 2, 16, 16
NW = NC * NS
C = 128
E_PAD = 327680
PER_W = E_PAD // NW
CHUNKS = PER_W // C
N_PAD = 10112
NBLK = N_PAD // C

_f32 = jnp.float32
_i32 = jnp.int32


def _k1_body(h_ref, w1a_ref, w1b_ref, be1_ref, a_ref, b_ref):
    h = h_ref[...]
    a_ref[...] = jnp.dot(h, w1a_ref[...], preferred_element_type=_f32) + be1_ref[...]
    b_ref[...] = jnp.dot(h, w1b_ref[...], preferred_element_type=_f32)


def _k1(h_pad, w1a, w1b, be1):
    blk = 1264
    grid = N_PAD // blk
    return pl.pallas_call(
        _k1_body,
        grid=(grid,),
        in_specs=[
            pl.BlockSpec((blk, D), lambda i: (i, 0)),
            pl.BlockSpec((D, D), lambda i: (0, 0)),
            pl.BlockSpec((D, D), lambda i: (0, 0)),
            pl.BlockSpec((1, D), lambda i: (0, 0)),
        ],
        out_specs=[
            pl.BlockSpec((blk, D), lambda i: (i, 0)),
            pl.BlockSpec((blk, D), lambda i: (i, 0)),
        ],
        out_shape=[
            jax.ShapeDtypeStruct((N_PAD, D), _f32),
            jax.ShapeDtypeStruct((N_PAD, D), _f32),
        ],
    )(h_pad, w1a, w1b, be1)


def _k2_body(a_hbm, b_hbm, src_hbm, dst_hbm, xm_hbm,
             p_hbm, xs_hbm, xd_hbm,
             buf_a, buf_b, src_v, dst_v, xs_v, xd_v,
             sem_a, sem_b, sem_c, sem_d):
    cid = lax.axis_index("c")
    sid = lax.axis_index("s")
    wid = sid * NC + cid
    base = pl.multiple_of(wid * PER_W, 8)

    def chunk(ci, carry):
        off = pl.multiple_of(base + ci * C, 8)
        pltpu.sync_copy(src_hbm.at[pl.ds(off, C)], src_v)
        pltpu.sync_copy(dst_hbm.at[pl.ds(off, C)], dst_v)
        cp_a = pltpu.async_copy(a_hbm.at[dst_v], buf_a, sem_a)
        cp_b = pltpu.async_copy(b_hbm.at[src_v], buf_b, sem_b)
        cp_c = pltpu.async_copy(xm_hbm.at[src_v], xs_v, sem_c)
        cp_d = pltpu.async_copy(xm_hbm.at[dst_v], xd_v, sem_d)
        cp_a.wait()
        cp_b.wait()

        def rowadd(r, c2):
            for cc in range(D // L):
                s2 = pl.ds(cc * L, L)
                buf_a[r, s2] = buf_a[r, s2] + buf_b[r, s2]
            return c2

        lax.fori_loop(0, C, rowadd, 0)
        cp_c.wait()
        cp_d.wait()

        pltpu.sync_copy(buf_a, p_hbm.at[pl.ds(off, C)])
        pltpu.sync_copy(xs_v, xs_hbm.at[pl.ds(off, C)])
        pltpu.sync_copy(xd_v, xd_hbm.at[pl.ds(off, C)])
        return carry

    lax.fori_loop(0, CHUNKS, chunk, 0)


def _k2(a, b, src_pad, dst_pad, xm8):
    mesh = plsc.VectorSubcoreMesh(core_axis_name="c", subcore_axis_name="s")
    return pl.kernel(
        _k2_body,
        out_type=[
            jax.ShapeDtypeStruct((E_PAD, D), _f32),
            jax.ShapeDtypeStruct((E_PAD, 8), _f32),
            jax.ShapeDtypeStruct((E_PAD, 8), _f32),
        ],
        mesh=mesh,
        scratch_types=[
            pltpu.VMEM((C, D), _f32),
            pltpu.VMEM((C, D), _f32),
            pltpu.VMEM((C,), _i32),
            pltpu.VMEM((C,), _i32),
            pltpu.VMEM((C, 8), _f32),
            pltpu.VMEM((C, 8), _f32),
            pltpu.SemaphoreType.DMA,
            pltpu.SemaphoreType.DMA,
            pltpu.SemaphoreType.DMA,
            pltpu.SemaphoreType.DMA,
        ],
        compiler_params=pltpu.CompilerParams(use_tc_tiling_on_sc=False),
    )(a, b, src_pad, dst_pad, xm8)


def _k3_body(p_ref, xs_ref, xd_ref, w1f_ref, we2_ref, be2_ref, winf_ref,
             binf_ref, wx1_ref, bx1_ref, wx2_ref, y_ref, v_ref):
    xs = xs_ref[...]
    xd = xd_ref[...]
    rel = xd[:, 0:3] - xs[:, 0:3]
    dsq = jnp.sum(rel * rel, axis=1, keepdims=True)
    et = 3.0 - 2.0 * xs[:, 3:4] - xd[:, 3:4]
    d = jnp.sqrt(dsq + 1e-8)
    step = 10.0 / 19.0
    sel = lax.broadcasted_iota(_i32, (1, 32), 1).astype(_f32)
    offs = sel * step
    coeff = -0.5 / (step * step)
    dfeat = jnp.exp(coeff * (d - offs) ** 2)
    oneh = (et == (sel - 20.0)).astype(_f32)
    keep = (sel < 20.0).astype(_f32)
    ef = dfeat * keep + oneh
    pre1 = p_ref[...] + jnp.dot(ef, w1f_ref[...], preferred_element_type=_f32)
    u = pre1 * jax.nn.sigmoid(pre1)
    mpre = jnp.dot(u, we2_ref[...], preferred_element_type=_f32) + be2_ref[...]
    mij = mpre * jax.nn.sigmoid(mpre)
    eij = jax.nn.sigmoid(
        jnp.dot(mij, winf_ref[...], preferred_element_type=_f32) + binf_ref[...])
    y_ref[...] = mij * eij
    t = jnp.dot(mij, wx1_ref[...], preferred_element_type=_f32) + bx1_ref[...]
    t = t * jax.nn.sigmoid(t)
    xw = jnp.tanh(jnp.dot(t, wx2_ref[...], preferred_element_type=_f32))
    s = xw / (d + 1.0)
    blk = rel.shape[0]
    v_ref[...] = jnp.concatenate([rel * s, jnp.zeros((blk, 5), _f32)], axis=1)


def _k3(p, xs, xd, w1f32, we2, be2, winf, binf, wx1, bx1, wx2):
    blk = 1024
    grid = E_PAD // blk
    cst = lambda i: (0, 0)
    return pl.pallas_call(
        _k3_body,
        grid=(grid,),
        in_specs=[
            pl.BlockSpec((blk, D), lambda i: (i, 0)),
            pl.BlockSpec((blk, 8), lambda i: (i, 0)),
            pl.BlockSpec((blk, 8), lambda i: (i, 0)),
            pl.BlockSpec((32, D), cst),
            pl.BlockSpec((D, D), cst),
            pl.BlockSpec((1, D), cst),
            pl.BlockSpec((D, 1), cst),
            pl.BlockSpec((1, 1), cst),
            pl.BlockSpec((D, D), cst),
            pl.BlockSpec((1, D), cst),
            pl.BlockSpec((D, 1), cst),
        ],
        out_specs=[
            pl.BlockSpec((blk, D), lambda i: (i, 0)),
            pl.BlockSpec((blk, 8), lambda i: (i, 0)),
        ],
        out_shape=[
            jax.ShapeDtypeStruct((E_PAD, D), _f32),
            jax.ShapeDtypeStruct((E_PAD, 8), _f32),
        ],
    )(p, xs, xd, w1f32, we2, be2, winf, binf, wx1, bx1, wx2)


def _k4_body(y_hbm, v_hbm, dst_hbm,
             mi_hbm, dx_hbm,
             y_v, v_v, dst_v, zero_v, y_tab, dx_tab, sem):
    cid = lax.axis_index("c")
    sid = lax.axis_index("s")
    wid = sid * NC + cid
    base = pl.multiple_of(wid * PER_W, 8)

    def zrow(r, c2):
        for cc in range(D // L):
            zero_v[r, pl.ds(cc * L, L)] = jnp.zeros((L,), _f32)
        return c2

    lax.fori_loop(0, C, zrow, 0)

    def zblk(j, c2):
        k = sid + j * NS

        @pl.when(k < NBLK)
        def _():
            pltpu.sync_copy(zero_v, y_tab.at[pl.ds(k * C, C)])
            pltpu.sync_copy(zero_v.at[:, pl.ds(0, 8)], dx_tab.at[pl.ds(k * C, C)])

        return c2

    lax.fori_loop(0, (NBLK + NS - 1) // NS, zblk, 0)
    plsc.subcore_barrier()

    def chunk(ci, carry):
        off = pl.multiple_of(base + ci * C, 8)
        pltpu.sync_copy(y_hbm.at[pl.ds(off, C)], y_v)
        pltpu.sync_copy(v_hbm.at[pl.ds(off, C)], v_v)
        pltpu.sync_copy(dst_hbm.at[pl.ds(off, C)], dst_v)
        pltpu.sync_copy(y_v, y_tab.at[dst_v], add=True)
        pltpu.sync_copy(v_v, dx_tab.at[dst_v], add=True)
        return carry

    lax.fori_loop(0, CHUNKS, chunk, 0)
    plsc.subcore_barrier()

    def out_blk(j, c2):
        k = sid + j * NS

        @pl.when(k < NBLK)
        def _():
            sl = pl.ds(k * C, C)
            pltpu.sync_copy(y_tab.at[sl], mi_hbm.at[cid].at[sl])
            pltpu.sync_copy(dx_tab.at[sl], dx_hbm.at[cid].at[sl])

        return c2

    lax.fori_loop(0, (NBLK + NS - 1) // NS, out_blk, 0)


def _k4(y, v, dst_pad):
    mesh = plsc.VectorSubcoreMesh(core_axis_name="c", subcore_axis_name="s")
    return pl.kernel(
        _k4_body,
        out_type=[
            jax.ShapeDtypeStruct((NC, N_PAD, D), _f32),
            jax.ShapeDtypeStruct((NC, N_PAD, 8), _f32),
        ],
        mesh=mesh,
        scratch_types=[
            pltpu.VMEM((C, D), _f32),
            pltpu.VMEM((C, 8), _f32),
            pltpu.VMEM((C,), _i32),
            pltpu.VMEM((C, D), _f32),
            pltpu.VMEM_SHARED((N_PAD, D), _f32),
            pltpu.VMEM_SHARED((N_PAD, 8), _f32),
            pltpu.SemaphoreType.DMA,
        ],
        compiler_params=pltpu.CompilerParams(use_tc_tiling_on_sc=False),
    )(y, v, dst_pad)


def _k5_body(mi0_ref, mi1_ref, h_ref, x4_ref, dx0_ref, dx1_ref, mk_ref,
             wn1a_ref, wn1b_ref, bn1_ref, wn2_ref, bn2_ref,
             hn_ref, xn_ref):
    mi = mi0_ref[...] + mi1_ref[...]
    h = h_ref[...]
    pre = (jnp.dot(mi, wn1a_ref[...], preferred_element_type=_f32)
           + jnp.dot(h, wn1b_ref[...], preferred_element_type=_f32)
           + bn1_ref[...])
    u = pre * jax.nn.sigmoid(pre)
    hn_ref[...] = h + jnp.dot(u, wn2_ref[...], preferred_element_type=_f32) + bn2_ref[...]
    dx = (dx0_ref[...] + dx1_ref[...])[:, 0:4]
    xn_ref[...] = x4_ref[...] + dx * mk_ref[...]


def _k5(mi0, mi1, h, x4n, dx0, dx1, mkf, wn1a, wn1b, bn1, wn2, bn2):
    blk = 2000
    grid = N // blk
    cst = lambda i: (0, 0)
    return pl.pallas_call(
        _k5_body,
        grid=(grid,),
        in_specs=[
            pl.BlockSpec((blk, D), lambda i: (i, 0)),
            pl.BlockSpec((blk, D), lambda i: (i, 0)),
            pl.BlockSpec((blk, D), lambda i: (i, 0)),
            pl.BlockSpec((blk, 4), lambda i: (i, 0)),
            pl.BlockSpec((blk, 8), lambda i: (i, 0)),
            pl.BlockSpec((blk, 8), lambda i: (i, 0)),
            pl.BlockSpec((blk, 1), lambda i: (i, 0)),
            pl.BlockSpec((D, D), cst),
            pl.BlockSpec((D, D), cst),
            pl.BlockSpec((1, D), cst),
            pl.BlockSpec((D, D), cst),
            pl.BlockSpec((1, D), cst),
        ],
        out_specs=[
            pl.BlockSpec((blk, D), lambda i: (i, 0)),
            pl.BlockSpec((blk, 4), lambda i: (i, 0)),
        ],
        out_shape=[
            jax.ShapeDtypeStruct((N, D), _f32),
            jax.ShapeDtypeStruct((N, 4), _f32),
        ],
    )(mi0, mi1, h, x4n, dx0, dx1, mkf, wn1a, wn1b, bn1, wn2, bn2)


def kernel(h, x, edge_index, mask_ligand, W_e1, b_e1, W_e2, b_e2, W_inf, b_inf,
           W_x1, b_x1, W_x2, W_n1, b_n1, W_n2, b_n2):
    src = edge_index[0]
    dst = edge_index[1]
    pad_e = E_PAD - E
    src_pad = jnp.concatenate([src, jnp.zeros((pad_e,), _i32)])
    dst_pad = jnp.concatenate([dst, jnp.full((pad_e,), N, _i32)])

    h_pad = jnp.concatenate([h, jnp.zeros((N_PAD - N, D), _f32)], axis=0)
    mkf = mask_ligand[:, None].astype(_f32)
    xm8 = jnp.concatenate([x, mkf, jnp.zeros((N, 4), _f32)], axis=1)
    xm8 = jnp.concatenate([xm8, jnp.zeros((N_PAD - N, 8), _f32)], axis=0)
    x4 = jnp.concatenate([x, jnp.zeros((N, 1), _f32)], axis=1)

    w1a = W_e1[0:D]
    w1b = W_e1[D:2 * D]
    w1f32 = jnp.concatenate([W_e1[2 * D:], jnp.zeros((8, D), _f32)], axis=0)
    be1 = b_e1.reshape(1, D)
    be2 = b_e2.reshape(1, D)
    binf = b_inf.reshape(1, 1)
    bx1 = b_x1.reshape(1, D)
    wn1a = W_n1[0:D]
    wn1b = W_n1[D:2 * D]
    bn1 = b_n1.reshape(1, D)
    bn2 = b_n2.reshape(1, D)

    a, b = _k1(h_pad, w1a, w1b, be1)
    p, xs, xd = _k2(a, b, src_pad, dst_pad, xm8)
    y, v = _k3(p, xs, xd, w1f32, W_e2, be2, W_inf, binf, W_x1, bx1, W_x2)
    mi_p, dx_p = _k4(y, v, dst_pad)

    mi0 = mi_p[0, :N]
    mi1 = mi_p[1, :N]
    dx0 = dx_p[0, :N]
    dx1 = dx_p[1, :N]

    hn, xn4 = _k5(mi0, mi1, h, x4, dx0, dx1, mkf, wn1a, wn1b, bn1, W_n2, bn2)
    return (hn, xn4[:, 0:3])
